# Initial kernel scaffold; baseline (speedup 1.0000x reference)
import jax
import jax.numpy as jnp
from jax.experimental import pallas as pl


def kernel(v, v_abs, W1, b1, gamma, beta, W2, b2):
    raise NotImplementedError("write your pallas kernel here")


# speedup vs baseline: 3696.9994x; 3696.9994x over previous
import functools

import jax
import jax.numpy as jnp
from jax import lax
from jax.experimental import pallas as pl
from jax.experimental.pallas import tpu as pltpu
from jax.experimental.pallas import tpu_sc as plsc

N = 512
TH = 1.0
TAU = 0.1
NCH = 32
L = 16
NCHUNK = N // L
WORDS = N // 16


def _tc_body(x_ref, xT_ref, v_ref, w1_ref, w1T_ref,
             w2eff_ref, b1_ref, shift_ref,
             vout_ref, ebits_ref, cmax_ref):
    x = x_ref[...]
    xT = xT_ref[...]
    v16 = v_ref[...]
    w1 = w1_ref[...]
    w1T = w1T_ref[...]

    y = jnp.dot(w1, x, preferred_element_type=jnp.float32,
                precision=lax.Precision.HIGHEST)
    yT = jnp.dot(xT, w1T, preferred_element_type=jnp.float32,
                 precision=lax.Precision.HIGHEST)

    acc = jnp.full((N, N), shift_ref[0], dtype=jnp.float32)
    for o in range(NCH):
        w = w2eff_ref[o]
        b = b1_ref[o]
        ycol = yT[:, o:o + 1] + b
        yrow = y[o:o + 1, :]
        acc = acc + w * jnp.maximum(ycol - yrow, 0.0)

    e_half = jnp.exp(acc)
    dm = 0.5 * (e_half + e_half.T)

    z = (TH - dm) * (1.0 / TAU)
    sig = 1.0 / (1.0 + jnp.exp(-z))
    colsum = jnp.sum(sig, axis=0, keepdims=True)
    sig_norm = sig / colsum
    v_soft = jnp.dot(v16, sig_norm, preferred_element_type=jnp.float32,
                     precision=lax.Precision.HIGHEST)
    vout_ref[...] = (v16 - v_soft) + v_soft

    ri = lax.broadcasted_iota(jnp.int32, (N, N), 0)
    ci = lax.broadcasted_iota(jnp.int32, (N, N), 1)
    e = (ci < ri) & (dm <= TH)
    cmax = jnp.max(jnp.where(e, ci, -1), axis=1, keepdims=True)
    cmax_ref[...] = cmax

    eprime = (e & (ci != cmax)).astype(jnp.float32)
    rc = lax.broadcasted_iota(jnp.int32, (N, WORDS), 0)
    wc = lax.broadcasted_iota(jnp.int32, (N, WORDS), 1)
    pmat = jnp.where((rc >> 4) == wc,
                     (jnp.int32(1) << (rc & 15)), 0).astype(jnp.float32)
    ebits_f = jnp.dot(eprime, pmat, preferred_element_type=jnp.float32,
                      precision=lax.Precision.HIGHEST)
    ebits_ref[...] = ebits_f.astype(jnp.int32)


def _run_tc(x, xT, v16, w1, w1T, w2eff, b1, shift):
    return pl.pallas_call(
        _tc_body,
        out_shape=(
            jax.ShapeDtypeStruct((L, N), jnp.float32),
            jax.ShapeDtypeStruct((N, WORDS), jnp.int32),
            jax.ShapeDtypeStruct((N, 1), jnp.int32),
        ),
        in_specs=[
            pl.BlockSpec((L, N), lambda: (0, 0)),
            pl.BlockSpec((N, L), lambda: (0, 0)),
            pl.BlockSpec((L, N), lambda: (0, 0)),
            pl.BlockSpec((NCH, L), lambda: (0, 0)),
            pl.BlockSpec((L, NCH), lambda: (0, 0)),
            pl.BlockSpec(memory_space=pltpu.SMEM),
            pl.BlockSpec(memory_space=pltpu.SMEM),
            pl.BlockSpec(memory_space=pltpu.SMEM),
        ],
    )(x, xT, v16, w1, w1T, w2eff, b1, shift)


def _sc_make():
    mesh = plsc.VectorSubcoreMesh(core_axis_name="c", subcore_axis_name="s")

    @functools.partial(
        pl.kernel, mesh=mesh,
        compiler_params=pltpu.CompilerParams(needs_layout_passes=False),
        out_type=jax.ShapeDtypeStruct((N,), jnp.int32),
        scratch_types=[
            pltpu.VMEM((N, WORDS), jnp.int32),
            pltpu.VMEM((N + L,), jnp.int32),
            pltpu.VMEM((N + L,), jnp.int32),
            pltpu.VMEM((N + L,), jnp.int32),
            pltpu.VMEM((N,), jnp.int32),
            pltpu.VMEM((N,), jnp.int32),
        ],
    )
    def sc_prog(ebits_hbm, cmax_hbm, out_hbm,
                ebits_v, cmax_v, labels_v, rowlist_v, rank_v, out_v):
        cid = lax.axis_index("c")
        sid = lax.axis_index("s")
        is_leader = jnp.logical_and(cid == 0, sid == 0)

        @pl.when(is_leader)
        def _():
            pltpu.sync_copy(ebits_hbm, ebits_v)
            pltpu.sync_copy(cmax_hbm, cmax_v.at[pl.ds(0, N)])

            lane = lax.iota(jnp.int32, L)

            def init_chunk(k, cnt):
                base = k * L
                basev = jnp.full((L,), base, dtype=jnp.int32)
                rows = lane + basev
                labels_v[pl.ds(base, L)] = rows
                av = cmax_v[pl.ds(base, L)] >= 0
                avi = av.astype(jnp.int32)
                pos = plsc.cumsum(avi) + jnp.full((L,), cnt - 1, jnp.int32)
                plsc.store_scatter(rowlist_v, [pos], rows, mask=av)
                return cnt + jnp.sum(avi)

            nrows = lax.fori_loop(0, NCHUNK, init_chunk, jnp.int32(0),
                                  unroll=False)

            def do_row(t, carry):
                r = rowlist_v[pl.ds(t, L)][0]
                cmax_r = cmax_v[pl.ds(r, L)][0]
                l0 = labels_v[pl.ds(r, L)][0]
                cmax_vec = jnp.full((L,), cmax_r, dtype=jnp.int32)
                l0_vec = jnp.full((L,), l0, dtype=jnp.int32)
                r_vec = jnp.full((L,), r, dtype=jnp.int32)

                def chunk(k, c2):
                    lab = labels_v[pl.ds(k * L, L)]
                    words = plsc.load_gather(ebits_v, [r_vec, lab >> 4])
                    bit = (words >> (lab & 15)) & 1
                    m = (bit != 0) | (lab == l0_vec)
                    labels_v[pl.ds(k * L, L)] = jnp.where(m, cmax_vec, lab)
                    return c2

                return lax.fori_loop(0, NCHUNK, chunk, carry, unroll=False)

            lax.fori_loop(0, nrows, do_row, jnp.int32(0), unroll=False)

            def zero_chunk(k, c):
                rank_v[pl.ds(k * L, L)] = jnp.zeros((L,), jnp.int32)
                return c
            lax.fori_loop(0, NCHUNK, zero_chunk, jnp.int32(0), unroll=False)

            ones = jnp.ones((L,), jnp.int32)

            def mark_chunk(k, c):
                lab = labels_v[pl.ds(k * L, L)]
                plsc.store_scatter(rank_v, [lab], ones)
                return c
            lax.fori_loop(0, NCHUNK, mark_chunk, jnp.int32(0), unroll=False)

            def rank_chunk(k, cnt):
                p = rank_v[pl.ds(k * L, L)]
                rank_v[pl.ds(k * L, L)] = (
                    plsc.cumsum(p) + jnp.full((L,), cnt - 1, jnp.int32))
                return cnt + jnp.sum(p)
            lax.fori_loop(0, NCHUNK, rank_chunk, jnp.int32(0), unroll=False)

            def out_chunk(k, c):
                lab = labels_v[pl.ds(k * L, L)]
                out_v[pl.ds(k * L, L)] = plsc.load_gather(rank_v, [lab])
                return c
            lax.fori_loop(0, NCHUNK, out_chunk, jnp.int32(0), unroll=False)

            pltpu.sync_copy(out_v, out_hbm)

    return sc_prog


_sc_prog = _sc_make()


def kernel(v, v_abs, W1, b1, gamma, beta, W2, b2):
    x = v_abs.reshape(L, N)
    v16 = v.reshape(L, N)
    w1 = W1[:, :, 0, 0]
    scale = gamma / jnp.sqrt(1.0 + 1e-5)
    w2 = W2[0, :, 0, 0]
    w2eff = w2 * scale
    shift = jnp.reshape(jnp.sum(w2 * beta) + b2[0], (1,))

    vout16, ebits, cmax2d = _run_tc(
        x, x.T, v16, w1, w1.T, w2eff, b1, shift)

    indices = _sc_prog(ebits, cmax2d.reshape(N))
    return (vout16.reshape(v.shape), indices)
